# Initial kernel scaffold; baseline (speedup 1.0000x reference)
import jax
import jax.numpy as jnp
from jax.experimental import pallas as pl


def kernel(iword, owords, nwords, W):
    raise NotImplementedError("write your pallas kernel here")


# speedup vs baseline: 1.3297x; 1.3297x over previous
import functools

import jax
import jax.numpy as jnp
from jax import lax
from jax.experimental import pallas as pl
from jax.experimental.pallas import tpu as pltpu
from jax.experimental.pallas import tpu_sc as plsc

_VOCAB = 1000000
_DIM = 64
_BATCH = 4096
_WINDOW = 20
_NEGS = 5
_BW = _BATCH * _WINDOW

_info = plsc.get_sparse_core_info()
_NC, _NS = _info.num_cores, _info.num_subcores
_NW = _NC * _NS
_CHUNK = _BATCH // _NW
_NROWS = _CHUNK * _NEGS


def _sc_scores(iword, oflat, nflat, W):
    mesh = plsc.VectorSubcoreMesh(core_axis_name="c", subcore_axis_name="s")

    @functools.partial(
        pl.kernel,
        mesh=mesh,
        out_type=jax.ShapeDtypeStruct((_BW * 6,), jnp.float32),
        scratch_types=[
            pltpu.VMEM((_CHUNK,), jnp.int32),
            pltpu.VMEM((_CHUNK,), jnp.int32),
            pltpu.VMEM((_NEGS, _CHUNK), jnp.int32),
            pltpu.VMEM((_CHUNK, _DIM), jnp.float32),
            pltpu.VMEM((_CHUNK, _DIM), jnp.float32),
            pltpu.VMEM((_NROWS, _DIM), jnp.float32),
            pltpu.VMEM((6, _CHUNK), jnp.float32),
            pltpu.SemaphoreType.DMA,
        ],
        compiler_params=pltpu.CompilerParams(
            needs_layout_passes=False, use_tc_tiling_on_sc=False),
    )
    def k(iword_h, oflat_h, nflat_h, w_h, out_h,
          iw_idx, o_idx, n_idx, iv_rows, ov_rows, ng_rows, sc_v, sem):
        wid = lax.axis_index("s") * _NC + lax.axis_index("c")
        base = wid * _CHUNK
        pltpu.sync_copy(iword_h.at[pl.ds(base, _CHUNK)], iw_idx)
        pltpu.async_copy(w_h.at[iw_idx], iv_rows, sem).wait()

        def jbody(j, carry):
            koff = j * _BATCH + base
            pltpu.sync_copy(oflat_h.at[pl.ds(koff, _CHUNK)], o_idx)
            for r in range(_NEGS):
                pltpu.sync_copy(
                    nflat_h.at[pl.ds(koff * _NEGS + r * _CHUNK, _CHUNK)],
                    n_idx.at[r])
            cps = [pltpu.async_copy(w_h.at[o_idx], ov_rows, sem)]
            for r in range(_NEGS):
                cps.append(pltpu.async_copy(
                    w_h.at[n_idx.at[r]],
                    ng_rows.at[pl.ds(r * _CHUNK, _CHUNK)], sem))
            for cp in cps:
                cp.wait()

            def gbody(g, c2):
                base16 = g * 16
                accs = [jnp.zeros((16,), jnp.float32) for _ in range(6)]
                for k16 in range(16):
                    kk = base16 + k16
                    lmask = jnp.arange(16, dtype=jnp.int32) == k16
                    iv0 = iv_rows[kk, pl.ds(0, 16)]
                    iv1 = iv_rows[kk, pl.ds(16, 16)]
                    iv2 = iv_rows[kk, pl.ds(32, 16)]
                    iv3 = iv_rows[kk, pl.ds(48, 16)]

                    def dot(ref, rr):
                        v = ref[rr, pl.ds(0, 16)] * iv0
                        v = v + ref[rr, pl.ds(16, 16)] * iv1
                        v = v + ref[rr, pl.ds(32, 16)] * iv2
                        v = v + ref[rr, pl.ds(48, 16)] * iv3
                        return jnp.sum(v)

                    accs[0] = jnp.where(lmask, dot(ov_rows, kk), accs[0])
                    for n in range(_NEGS):
                        accs[1 + n] = jnp.where(
                            lmask, -dot(ng_rows, kk * _NEGS + n), accs[1 + n])
                for t in range(6):
                    sc_v[t, pl.ds(base16, 16)] = accs[t]
                return c2

            lax.fori_loop(0, _CHUNK // 16, gbody, 0)
            for t in range(6):
                pltpu.sync_copy(sc_v.at[t],
                                out_h.at[pl.ds(t * _BW + koff, _CHUNK)])
            return carry

        lax.fori_loop(0, _WINDOW, jbody, 0)

    return k(iword, oflat, nflat, W)


_TC_ROWS = 384
_TC_COLS = (_BW * 6) // _TC_ROWS


def _tc_loss(scores):
    def body(s_ref, o_ref):
        x = s_ref[...]
        o_ref[...] = jnp.reshape(
            -jnp.sum(jnp.log(jax.nn.sigmoid(x))) / _BATCH, (1, 1))

    return pl.pallas_call(
        body,
        out_shape=jax.ShapeDtypeStruct((1, 1), jnp.float32),
    )(scores.reshape(_TC_ROWS, _TC_COLS))


def kernel(iword, owords, nwords, W):
    scores = _sc_scores(iword, owords.reshape(-1), nwords.reshape(-1), W)
    return _tc_loss(scores)[0, 0]


# speedup vs baseline: 1.4425x; 1.0848x over previous
import functools

import jax
import jax.numpy as jnp
from jax import lax
from jax.experimental import pallas as pl
from jax.experimental.pallas import tpu as pltpu
from jax.experimental.pallas import tpu_sc as plsc

_VOCAB = 1000000
_DIM = 64
_BATCH = 4096
_WINDOW = 20
_NEGS = 5
_BW = _BATCH * _WINDOW

_info = plsc.get_sparse_core_info()
_NC, _NS = _info.num_cores, _info.num_subcores
_NW = _NC * _NS
_CHUNK = _BATCH // _NW


def _sc_scores(iword, oflat, nwords, W):
    mesh = plsc.VectorSubcoreMesh(core_axis_name="c", subcore_axis_name="s")

    @functools.partial(
        pl.kernel,
        mesh=mesh,
        out_type=jax.ShapeDtypeStruct((6 * _BW,), jnp.float32),
        scratch_types=[
            pltpu.VMEM((_CHUNK,), jnp.int32),
            [pltpu.VMEM((_CHUNK,), jnp.int32)] * 2,
            [pltpu.VMEM((_CHUNK, _NEGS), jnp.int32)] * 2,
            [pltpu.VMEM((_NEGS * _CHUNK,), jnp.int32)] * 2,
            pltpu.VMEM((_CHUNK, _DIM), jnp.float32),
            [pltpu.VMEM((_CHUNK, _DIM), jnp.float32)] * 2,
            [pltpu.VMEM((_NEGS, _CHUNK, _DIM), jnp.float32)] * 2,
            pltpu.VMEM((6, _CHUNK), jnp.float32),
            [pltpu.SemaphoreType.DMA] * 2,
        ],
        compiler_params=pltpu.CompilerParams(
            needs_layout_passes=False, use_tc_tiling_on_sc=False),
    )
    def k(iword_h, oflat_h, nwords_h, w_h, out_h,
          iw_idx, o_idx, n2_idx, n_idx, iv_rows, ov_rows, ng_rows, sc_v, sem):
        wid = lax.axis_index("s") * _NC + lax.axis_index("c")
        base = wid * _CHUNK
        pltpu.sync_copy(iword_h.at[pl.ds(base, _CHUNK)], iw_idx)
        pltpu.async_copy(w_h.at[iw_idx], iv_rows, sem[0]).wait()

        def issue(j, p):
            koff = j * _BATCH + base
            pltpu.sync_copy(oflat_h.at[pl.ds(koff, _CHUNK)], o_idx[p])
            pltpu.sync_copy(nwords_h.at[pl.ds(koff, _CHUNK), :], n2_idx[p])
            pltpu.async_copy(w_h.at[o_idx[p]], ov_rows[p], sem[p])
            iota = lax.iota(jnp.int32, 16)
            for n in range(_NEGS):
                for g in range(_CHUNK // 16):
                    v = plsc.load_gather(
                        n2_idx[p], [g * 16 + iota, jnp.full((16,), n, jnp.int32)])
                    n_idx[p][pl.ds(n * _CHUNK + g * 16, 16)] = v
            for n in range(_NEGS):
                pltpu.async_copy(
                    w_h.at[n_idx[p].at[pl.ds(n * _CHUNK, _CHUNK)]],
                    ng_rows[p].at[n], sem[p])

        def wait(p):
            pltpu.make_async_copy(w_h.at[o_idx[p]], ov_rows[p], sem[p]).wait()
            for n in range(_NEGS):
                pltpu.make_async_copy(
                    w_h.at[n_idx[p].at[pl.ds(n * _CHUNK, _CHUNK)]],
                    ng_rows[p].at[n], sem[p]).wait()

        def compute(j, p):
            koff = j * _BATCH + base
            ov = ov_rows[p]
            ng = ng_rows[p]

            def gbody(g, c2):
                base16 = g * 16
                accs = [jnp.zeros((16,), jnp.float32) for _ in range(6)]
                for k16 in range(16):
                    kk = base16 + k16
                    lmask = jnp.arange(16, dtype=jnp.int32) == k16
                    iv0 = iv_rows[kk, pl.ds(0, 16)]
                    iv1 = iv_rows[kk, pl.ds(16, 16)]
                    iv2 = iv_rows[kk, pl.ds(32, 16)]
                    iv3 = iv_rows[kk, pl.ds(48, 16)]

                    def dot(r0, r1, r2, r3):
                        v = r0 * iv0
                        v = v + r1 * iv1
                        v = v + r2 * iv2
                        v = v + r3 * iv3
                        return jnp.sum(v)

                    accs[0] = jnp.where(
                        lmask,
                        dot(ov[kk, pl.ds(0, 16)], ov[kk, pl.ds(16, 16)],
                            ov[kk, pl.ds(32, 16)], ov[kk, pl.ds(48, 16)]),
                        accs[0])
                    for n in range(_NEGS):
                        accs[1 + n] = jnp.where(
                            lmask,
                            -dot(ng[n, kk, pl.ds(0, 16)],
                                 ng[n, kk, pl.ds(16, 16)],
                                 ng[n, kk, pl.ds(32, 16)],
                                 ng[n, kk, pl.ds(48, 16)]),
                            accs[1 + n])
                for t in range(6):
                    sc_v[t, pl.ds(base16, 16)] = accs[t]
                return c2

            lax.fori_loop(0, _CHUNK // 16, gbody, 0)
            for t in range(6):
                pltpu.sync_copy(sc_v.at[t],
                                out_h.at[pl.ds(t * _BW + koff, _CHUNK)])

        issue(0, 0)

        def jbody(jj, carry):
            j0 = jj * 2
            issue(j0 + 1, 1)
            wait(0)
            compute(j0, 0)

            @pl.when(jj + 1 < _WINDOW // 2)
            def _():
                issue(j0 + 2, 0)

            wait(1)
            compute(j0 + 1, 1)
            return carry

        lax.fori_loop(0, _WINDOW // 2, jbody, 0)

    return k(iword, oflat, nwords, W)


_TC_ROWS = 384
_TC_COLS = (6 * _BW) // _TC_ROWS


def _tc_loss(scores):
    def body(s_ref, o_ref):
        x = s_ref[...]
        o_ref[...] = jnp.reshape(
            -jnp.sum(jnp.log(jax.nn.sigmoid(x))) / _BATCH, (1, 1))

    return pl.pallas_call(
        body,
        out_shape=jax.ShapeDtypeStruct((1, 1), jnp.float32),
    )(scores.reshape(_TC_ROWS, _TC_COLS))


def kernel(iword, owords, nwords, W):
    scores = _sc_scores(iword, owords.reshape(-1), nwords, W)
    return _tc_loss(scores)[0, 0]
